# baseline (speedup 1.0000x reference)
import jax
import jax.numpy as jnp
from jax.experimental import pallas as pl
from jax.experimental.pallas import tpu as pltpu

N_NODES = 100000
K = 10
ALPHA = 0.1

_BLK = 4000


def _mlp_body(x_ref, w1_ref, b1_ref, w2_ref, b2_ref, o_ref):
    h = jnp.dot(x_ref[...], w1_ref[...], preferred_element_type=jnp.float32)
    h = jnp.maximum(h + b1_ref[...], 0.0)
    o_ref[...] = jnp.dot(h, w2_ref[...], preferred_element_type=jnp.float32) + b2_ref[...]


def _mlp(x, W1, b1, W2, b2):
    n, f = x.shape
    neurons = W1.shape[1]
    classes = W2.shape[1]
    grid = (n // _BLK,)
    return pl.pallas_call(
        _mlp_body,
        grid=grid,
        in_specs=[
            pl.BlockSpec((_BLK, f), lambda i: (i, 0)),
            pl.BlockSpec((f, neurons), lambda i: (0, 0)),
            pl.BlockSpec((1, neurons), lambda i: (0, 0)),
            pl.BlockSpec((neurons, classes), lambda i: (0, 0)),
            pl.BlockSpec((1, classes), lambda i: (0, 0)),
        ],
        out_specs=pl.BlockSpec((_BLK, classes), lambda i: (i, 0)),
        out_shape=jax.ShapeDtypeStruct((n, classes), jnp.float32),
    )(x, W1, b1.reshape(1, -1), W2, b2.reshape(1, -1))


def kernel(x, edge_index, W1, b1, W2, b2):
    h = _mlp(x, W1, b1, W2, b2)

    n = x.shape[0]
    loop = jnp.arange(n, dtype=edge_index.dtype)
    src = jnp.concatenate([edge_index[0], loop])
    dst = jnp.concatenate([edge_index[1], loop])
    deg = jax.ops.segment_sum(jnp.ones_like(src, dtype=h.dtype), dst, num_segments=n)
    dinv = jnp.where(deg > 0, jax.lax.rsqrt(deg), 0.0)
    norm = dinv[src] * dinv[dst]

    h0 = h
    for _ in range(K):
        msg = norm[:, None] * jnp.take(h, src, axis=0)
        agg = jax.ops.segment_sum(msg, dst, num_segments=n)
        h = (1.0 - ALPHA) * agg + ALPHA * h0
    return jax.nn.log_softmax(h, axis=1)


# speedup vs baseline: 38.0690x; 38.0690x over previous
import functools

import jax
import jax.numpy as jnp
from jax import lax
from jax.experimental import pallas as pl
from jax.experimental.pallas import tpu as pltpu
from jax.experimental.pallas import tpu_sc as plsc

N = 100000
NP = 102400
E = 3200000
CLS = 16
K = 10
ALPHA = 0.1

NC, NS = 2, 16
NW = NC * NS
EPT = E // NW
CHUNK = 125
BC = 8
BLK_E = CHUNK * BC
NB = EPT // BLK_E
ROWS_PT = NP // NS
ZROWS = 400

_TC_BLK = 3200
_OUT_BLK = 4000


def _edge_body(g_hbm, src_hbm, dst_hbm, out_hbm,
               s_sp, sidx, didx, rows, zbuf, gsem, isem):
    c = lax.axis_index("c")
    s = lax.axis_index("s")
    w = s * NC + c
    base = pl.multiple_of(s * ROWS_PT, 8)

    def _zstore(i, carry):
        zbuf[i] = jnp.zeros((CLS,), jnp.float32)
        return carry

    lax.fori_loop(0, ZROWS, _zstore, 0)
    for kk in range(ROWS_PT // ZROWS):
        pltpu.sync_copy(zbuf, s_sp.at[pl.ds(base + kk * ZROWS, ZROWS)])
    plsc.subcore_barrier()

    pltpu.async_copy(src_hbm.at[w, 0], sidx.at[0], isem)
    pltpu.async_copy(dst_hbm.at[w, 0], didx.at[0], isem)

    def _block(b, carry):
        cur = lax.rem(b, 2)
        nxt = lax.rem(b + 1, 2)
        pltpu.make_async_copy(src_hbm.at[w, 0], sidx.at[cur], isem).wait()
        pltpu.make_async_copy(dst_hbm.at[w, 0], didx.at[cur], isem).wait()

        @pl.when(b + 1 < NB)
        def _prefetch():
            pltpu.async_copy(src_hbm.at[w, b + 1], sidx.at[nxt], isem)
            pltpu.async_copy(dst_hbm.at[w, b + 1], didx.at[nxt], isem)

        descs = [
            pltpu.async_copy(g_hbm.at[sidx.at[cur, j]], rows.at[j], gsem)
            for j in range(BC)
        ]
        for d in descs:
            d.wait()
        for j in range(BC):
            pltpu.sync_copy(rows.at[j], s_sp.at[didx.at[cur, j]], add=True)
        return carry

    lax.fori_loop(0, NB, _block, 0)

    plsc.subcore_barrier()
    pltpu.sync_copy(s_sp.at[pl.ds(base, ROWS_PT)],
                    out_hbm.at[c, pl.ds(base, ROWS_PT)])


_edge_pass = functools.partial(
    pl.kernel,
    out_type=jax.ShapeDtypeStruct((NC, NP, CLS), jnp.float32),
    mesh=plsc.VectorSubcoreMesh(core_axis_name="c", subcore_axis_name="s"),
    compiler_params=pltpu.CompilerParams(use_tc_tiling_on_sc=False),
    scratch_types=[
        pltpu.VMEM_SHARED((NP, CLS), jnp.float32),
        pltpu.VMEM((2, BC, CHUNK), jnp.int32),
        pltpu.VMEM((2, BC, CHUNK), jnp.int32),
        pltpu.VMEM((BC, CHUNK, CLS), jnp.float32),
        pltpu.VMEM((ZROWS, CLS), jnp.float32),
        pltpu.SemaphoreType.DMA,
        pltpu.SemaphoreType.DMA,
    ],
)(_edge_body)


def _mlp_body(x_ref, w1_ref, b1_ref, w2_ref, b2_ref, o_ref):
    h = jnp.dot(x_ref[...], w1_ref[...], preferred_element_type=jnp.float32)
    h = jnp.maximum(h + b1_ref[...], 0.0)
    o_ref[...] = jnp.dot(h, w2_ref[...], preferred_element_type=jnp.float32) + b2_ref[...]


def _mlp(x, W1, b1, W2, b2):
    n, f = x.shape
    neu = W1.shape[1]
    return pl.pallas_call(
        _mlp_body,
        grid=(n // _OUT_BLK,),
        in_specs=[
            pl.BlockSpec((_OUT_BLK, f), lambda i: (i, 0)),
            pl.BlockSpec((f, neu), lambda i: (0, 0)),
            pl.BlockSpec((1, neu), lambda i: (0, 0)),
            pl.BlockSpec((neu, CLS), lambda i: (0, 0)),
            pl.BlockSpec((1, CLS), lambda i: (0, 0)),
        ],
        out_specs=pl.BlockSpec((_OUT_BLK, CLS), lambda i: (i, 0)),
        out_shape=jax.ShapeDtypeStruct((n, CLS), jnp.float32),
    )(x, W1, b1.reshape(1, -1), W2, b2.reshape(1, -1))


def _rowblock_call(body, n_out, args, blk, n_rows):
    in_specs = []
    ops = []
    for a in args:
        if a.ndim == 3:
            for kkk in range(NC):
                in_specs.append(
                    pl.BlockSpec((1, blk, CLS), lambda i, _k=kkk: (_k, i, 0)))
                ops.append(a)
        else:
            in_specs.append(pl.BlockSpec((blk, CLS), lambda i: (i, 0)))
            ops.append(a)
    outs = [jax.ShapeDtypeStruct((n_rows, CLS), jnp.float32)] * n_out
    out_specs = [pl.BlockSpec((blk, CLS), lambda i: (i, 0))] * n_out
    return pl.pallas_call(
        body,
        grid=(n_rows // blk,),
        in_specs=in_specs,
        out_specs=out_specs[0] if n_out == 1 else out_specs,
        out_shape=outs[0] if n_out == 1 else outs,
    )(*ops)


def _prep_body(s0_ref, s1_ref, h0_ref, a_ref, c_ref, g_ref):
    deg = s0_ref[0] + s1_ref[0] + 1.0
    dinv = lax.rsqrt(deg)
    h0 = h0_ref[...]
    a_ref[...] = (1.0 - ALPHA) * dinv * dinv
    c_ref[...] = ALPHA * dinv * h0
    g_ref[...] = dinv * h0


def _iter_body(s0_ref, s1_ref, g_ref, a_ref, c_ref, o_ref):
    stot = s0_ref[0] + s1_ref[0] + g_ref[...]
    o_ref[...] = a_ref[...] * stot + c_ref[...]


def _final_body(s0_ref, s1_ref, g_ref, a_ref, c_ref, o_ref):
    dinv = jnp.sqrt(a_ref[...] * (1.0 / (1.0 - ALPHA)))
    stot = s0_ref[0] + s1_ref[0] + g_ref[...]
    h = (1.0 - ALPHA) * dinv * stot + c_ref[...] / dinv
    m = jnp.max(h, axis=1, keepdims=True)
    e = jnp.exp(h - m)
    lse = jnp.log(jnp.sum(e, axis=1, keepdims=True))
    o_ref[...] = h - m - lse


def kernel(x, edge_index, W1, b1, W2, b2):
    h0 = _mlp(x, W1, b1, W2, b2)
    h0p = jnp.pad(h0, ((0, NP - N), (0, 0)))

    ei = edge_index.astype(jnp.int32)
    src4 = ei[0].reshape(NW, NB, BC, CHUNK)
    dst4 = ei[1].reshape(NW, NB, BC, CHUNK)

    ones = jnp.ones((NP, CLS), jnp.float32)
    sdeg = _edge_pass(ones, src4, dst4)
    a, cc, g = _rowblock_call(_prep_body, 3, [sdeg, h0p], _TC_BLK, NP)

    for t in range(K):
        s_pair = _edge_pass(g, src4, dst4)
        if t < K - 1:
            g = _rowblock_call(_iter_body, 1, [s_pair, g, a, cc], _TC_BLK, NP)
        else:
            out = _rowblock_call(_final_body, 1, [s_pair, g, a, cc],
                                 _OUT_BLK, N)
    return out
